# Initial kernel scaffold; baseline (speedup 1.0000x reference)
import jax
import jax.numpy as jnp
from jax.experimental import pallas as pl


def kernel(x, edge_index, W1, b1, W2, b2):
    raise NotImplementedError("write your pallas kernel here")


# speedup vs baseline: 10.9730x; 10.9730x over previous
import jax
import jax.numpy as jnp
from jax import lax
from jax.experimental import pallas as pl
from jax.experimental.pallas import tpu as pltpu
from jax.experimental.pallas import tpu_sc as plsc

N = 10000
NPAD = 10240
E = 320000
FW = 128
CHUNK = 128
NC, NS = 2, 16
NW = NC * NS
EPW = ((E + NW * CHUNK - 1) // (NW * CHUNK)) * CHUNK
EPAD = EPW * NW
PADIDX = N
RPT = NPAD // NS


_MESH = plsc.VectorSubcoreMesh(core_axis_name="c", subcore_axis_name="s")


def _deg_body(dst_hbm, out_hbm, deg_s, idx, ones, zbuf, sem):
    c = lax.axis_index("c")
    s = lax.axis_index("s")
    wid = c * NS + s
    for i in range(RPT // 16):
        zbuf[pl.ds(i * 16, 16)] = jnp.zeros((16,), jnp.float32)
    for i in range(CHUNK // 16):
        ones[pl.ds(i * 16, 16)] = jnp.ones((16,), jnp.float32)
    pltpu.sync_copy(zbuf, deg_s.at[pl.ds(s * RPT, RPT)])
    plsc.subcore_barrier()

    def body(j, carry):
        base = wid * EPW + j * CHUNK
        pltpu.sync_copy(dst_hbm.at[pl.ds(base, CHUNK)], idx)
        pltpu.sync_copy(ones, deg_s.at[idx], add=True)
        return carry

    lax.fori_loop(0, EPW // CHUNK, body, 0)
    plsc.subcore_barrier()
    pltpu.sync_copy(deg_s.at[pl.ds(s * RPT, RPT)],
                    out_hbm.at[c].at[pl.ds(s * RPT, RPT)])


def _run_deg(dst):
    kern = pl.kernel(
        _deg_body,
        out_type=jax.ShapeDtypeStruct((NC, NPAD), jnp.float32),
        mesh=_MESH,
        scratch_types=[
            pltpu.VMEM_SHARED((NPAD,), jnp.float32),
            pltpu.VMEM((CHUNK,), jnp.int32),
            pltpu.VMEM((CHUNK,), jnp.float32),
            pltpu.VMEM((RPT,), jnp.float32),
            pltpu.SemaphoreType.DMA,
        ],
    )
    return kern(dst)


def _agg_body(src_hbm, dst_hbm, feat_hbm, out_hbm, acc, idx_s, idx_d, rows, sem):
    c = lax.axis_index("c")
    s = lax.axis_index("s")
    wid = c * NS + s
    pltpu.sync_copy(feat_hbm.at[pl.ds(s * RPT, RPT)], acc.at[pl.ds(s * RPT, RPT)])
    plsc.subcore_barrier()

    def body(j, carry):
        base = wid * EPW + j * CHUNK
        pltpu.sync_copy(src_hbm.at[pl.ds(base, CHUNK)], idx_s)
        pltpu.sync_copy(dst_hbm.at[pl.ds(base, CHUNK)], idx_d)
        pltpu.async_copy(feat_hbm.at[idx_s], rows, sem).wait()
        pltpu.sync_copy(rows, acc.at[idx_d], add=True)
        return carry

    lax.fori_loop(0, EPW // CHUNK, body, 0)
    plsc.subcore_barrier()
    pltpu.sync_copy(acc.at[pl.ds(s * RPT, RPT)],
                    out_hbm.at[c].at[pl.ds(s * RPT, RPT)])


def _run_agg(src, dst, feat):
    kern = pl.kernel(
        _agg_body,
        out_type=jax.ShapeDtypeStruct((NC, NPAD, FW), jnp.float32),
        mesh=_MESH,
        scratch_types=[
            pltpu.VMEM_SHARED((NPAD, FW), jnp.float32),
            pltpu.VMEM((CHUNK,), jnp.int32),
            pltpu.VMEM((CHUNK,), jnp.int32),
            pltpu.VMEM((CHUNK, FW), jnp.float32),
            pltpu.SemaphoreType.DMA,
        ],
    )
    return kern(src, dst, feat)


def _h_body(x_ref, w1_ref, degt_ref, hs_ref, dinv_ref):
    d = degt_ref[:, 0:1] + degt_ref[:, 1:2] + 1.0
    dinv = lax.rsqrt(d)
    dinv_ref[...] = dinv
    h = jnp.dot(x_ref[...], w1_ref[...], preferred_element_type=jnp.float32)
    hs_ref[...] = h * dinv


def _run_h(xp, w1p, degt):
    return pl.pallas_call(
        _h_body,
        out_shape=(
            jax.ShapeDtypeStruct((NPAD, FW), jnp.float32),
            jax.ShapeDtypeStruct((NPAD, 1), jnp.float32),
        ),
    )(xp, w1p, degt)


def _mid_body(aggp_ref, hs_ref, dinv_ref, b1_ref, gs_ref):
    ssum = aggp_ref[0] + aggp_ref[1] - hs_ref[...]
    dinv = dinv_ref[...]
    pre = ssum * dinv + b1_ref[...]
    gs_ref[...] = jax.nn.sigmoid(pre) * dinv


def _run_mid(aggp, hs, dinv, b1p):
    return pl.pallas_call(
        _mid_body,
        out_shape=jax.ShapeDtypeStruct((NPAD, FW), jnp.float32),
    )(aggp, hs, dinv, b1p)


def _out_body(aggp_ref, gs_ref, dinv_ref, w2p_ref, b2_ref, z_ref):
    ssum = aggp_ref[0] + aggp_ref[1] - gs_ref[...]
    zin = ssum * dinv_ref[...]
    z = jnp.dot(zin, w2p_ref[...], preferred_element_type=jnp.float32)
    z_ref[...] = z[0:N, :] + b2_ref[...]


def _run_out(aggp, gs, dinv, w2p, b2):
    return pl.pallas_call(
        _out_body,
        out_shape=jax.ShapeDtypeStruct((N, 8), jnp.float32),
    )(aggp, gs, dinv, w2p, b2)


def kernel(x, edge_index, W1, b1, W2, b2):
    ei = edge_index.astype(jnp.int32)
    pad = jnp.full((EPAD - E,), PADIDX, dtype=jnp.int32)
    src = jnp.concatenate([ei[0], pad])
    dst = jnp.concatenate([ei[1], pad])
    xp = jnp.pad(x, ((0, NPAD - N), (0, 0)))
    w1p = jnp.pad(W1, ((0, 0), (0, FW - W1.shape[1])))
    w2p = jnp.pad(W2, ((0, FW - W2.shape[0]), (0, 0)))
    b1p = jnp.pad(b1, (0, FW - b1.shape[0])).reshape(1, FW)
    b2r = b2.reshape(1, -1)

    degp = _run_deg(dst)
    hs, dinv = _run_h(xp, w1p, degp.T)
    aggp = _run_agg(src, dst, hs)
    gs = _run_mid(aggp, hs, dinv, b1p)
    agg2p = _run_agg(src, dst, gs)
    return _run_out(agg2p, gs, dinv, w2p, b2r)
